# baseline (speedup 1.0000x reference)
import functools

import jax
import jax.numpy as jnp
from jax import lax
from jax.experimental import pallas as pl
from jax.experimental.pallas import tpu as pltpu
from jax.experimental.pallas import tpu_sc as plsc

N_NODES = 100000
D_STRUCT = 131
D_SPATIAL = 64
D_OUT = 256

NC, NS = 2, 16
NW = NC * NS
B_PER_W = 3200
N_IDX = NW * B_PER_W
CHUNK = 40
N_CH = B_PER_W // CHUNK
SLICES = D_OUT // 16

TC_BLOCK = 512


def _sc_gather_mean(table, idx_r):
    mesh = plsc.VectorSubcoreMesh(core_axis_name="c", subcore_axis_name="s")
    idx_per_w = N_CH * 3 * CHUNK

    @functools.partial(
        pl.kernel,
        out_type=jax.ShapeDtypeStruct((N_NODES, D_OUT), jnp.float32),
        mesh=mesh,
        scratch_types=[
            pltpu.VMEM((idx_per_w,), jnp.int32),
            [pltpu.VMEM((3 * CHUNK, D_OUT), jnp.float32) for _ in range(2)],
            [pltpu.VMEM((CHUNK, D_OUT), jnp.float32) for _ in range(2)],
            [pltpu.SemaphoreType.DMA for _ in range(2)],
            [pltpu.SemaphoreType.DMA for _ in range(2)],
        ],
    )
    def k(table_hbm, idx_hbm, out_hbm, iv, g, acc, sems, osems):
        wid = lax.axis_index("s") * NC + lax.axis_index("c")
        wbase = wid * B_PER_W
        n_chunks = jnp.where(wid == NW - 1, 800 // CHUNK, B_PER_W // CHUNK)
        pltpu.sync_copy(idx_hbm.at[pl.ds(wid * idx_per_w, idx_per_w)], iv)

        def issue2(c, b):
            dg = pltpu.async_copy(
                table_hbm.at[iv.at[pl.ds(c * 3 * CHUNK, 3 * CHUNK)]],
                g[b], sems[b])
            ds = pltpu.async_copy(
                table_hbm.at[pl.ds(wbase + c * CHUNK, CHUNK)], acc[b], sems[b])
            return (dg, ds)

        def compute_and_store(c, b):
            def row_body(j, c2):
                for d in range(SLICES):
                    sl = pl.ds(d * 16, 16)
                    acc[b][j, sl] = (
                        acc[b][j, sl] + g[b][j, sl] + g[b][CHUNK + j, sl]
                        + g[b][2 * CHUNK + j, sl]
                    ) * 0.25
                return c2

            lax.fori_loop(0, CHUNK, row_body, 0, unroll=2)
            pltpu.async_copy(
                acc[b], out_hbm.at[pl.ds(wbase + c * CHUNK, CHUNK)], osems[b])

        def wait_out(c, b):
            pltpu.make_async_copy(
                acc[b], out_hbm.at[pl.ds(wbase + c * CHUNK, CHUNK)],
                osems[b]).wait()

        def pair_body(i, carry):
            c0 = 2 * i

            @pl.when(c0 >= 2)
            def _():
                wait_out(c0 - 2, 0)

            da = issue2(c0, 0)

            @pl.when(c0 >= 2)
            def _():
                wait_out(c0 - 1, 1)

            db = issue2(c0 + 1, 1)
            for d in da:
                d.wait()
            compute_and_store(c0, 0)
            for d in db:
                d.wait()
            compute_and_store(c0 + 1, 1)
            return carry

        lax.fori_loop(0, n_chunks // 2, pair_body, 0)
        wait_out(n_chunks - 2, 0)
        wait_out(n_chunks - 1, 1)

    return k(table, idx_r)


def _p_body(stt_ref, wa_ref, ba_ref, p_ref):
    p_ref[...] = (
        lax.dot_general(
            stt_ref[...], wa_ref[...],
            dimension_numbers=(((0,), (0,)), ((), ())),
            preferred_element_type=jnp.float32,
        )
        + ba_ref[...]
    )


def _tc_p(structural_t, WaT, b_agg):
    grid = (pl.cdiv(N_NODES, TC_BLOCK),)
    full = lambda i: (0, 0)
    return pl.pallas_call(
        _p_body,
        grid=grid,
        in_specs=[
            pl.BlockSpec((D_STRUCT, TC_BLOCK), lambda i: (0, i)),
            pl.BlockSpec((D_STRUCT, D_OUT), full),
            pl.BlockSpec((1, D_OUT), full),
        ],
        out_specs=pl.BlockSpec((TC_BLOCK, D_OUT), lambda i: (i, 0)),
        out_shape=jax.ShapeDtypeStruct((N_NODES, D_OUT), jnp.float32),
    )(structural_t, WaT, b_agg)


def _out1_body(spt_ref, stt_ref, wc_ref, bc_ref, o1_ref):
    dn = (((0,), (0,)), ((), ()))
    wc = wc_ref[...]
    o1_ref[...] = (
        lax.dot_general(spt_ref[...], wc[:D_SPATIAL], dimension_numbers=dn,
                        preferred_element_type=jnp.float32)
        + lax.dot_general(stt_ref[...], wc[D_SPATIAL:], dimension_numbers=dn,
                          preferred_element_type=jnp.float32)
        + bc_ref[...]
    )


def _tc_out1(spatial_t, structural_t, WcT, b_comb):
    grid = (pl.cdiv(N_NODES, TC_BLOCK),)
    full = lambda i: (0, 0)
    return pl.pallas_call(
        _out1_body,
        grid=grid,
        in_specs=[
            pl.BlockSpec((D_SPATIAL, TC_BLOCK), lambda i: (0, i)),
            pl.BlockSpec((D_STRUCT, TC_BLOCK), lambda i: (0, i)),
            pl.BlockSpec((D_SPATIAL + D_STRUCT, D_OUT), full),
            pl.BlockSpec((1, D_OUT), full),
        ],
        out_specs=pl.BlockSpec((TC_BLOCK, D_OUT), lambda i: (i, 0)),
        out_shape=jax.ShapeDtypeStruct((N_NODES, D_OUT), jnp.float32),
    )(spatial_t, structural_t, WcT, b_comb)


def kernel(spatial, structural, neighbour, W_comb, b_comb, W_agg, b_agg):
    nb32 = neighbour.astype(jnp.int32)
    nbp = jnp.pad(nb32, ((0, N_IDX - N_NODES), (0, 0)))
    idx_r = nbp.reshape(NW, N_CH, CHUNK, 3).transpose(0, 1, 3, 2).reshape(-1)

    WcT = W_comb.T
    WaT = W_agg.T
    spatial_t = spatial.T
    structural_t = structural.T

    P = _tc_p(structural_t, WaT, b_agg.reshape(1, D_OUT))
    out2 = _sc_gather_mean(P, idx_r)
    out1 = _tc_out1(spatial_t, structural_t, WcT, b_comb.reshape(1, D_OUT))
    return (out1, out2)


# speedup vs baseline: 1.2527x; 1.2527x over previous
import functools

import jax
import jax.numpy as jnp
from jax import lax
from jax.experimental import pallas as pl
from jax.experimental.pallas import tpu as pltpu
from jax.experimental.pallas import tpu_sc as plsc

N_NODES = 100000
D_STRUCT = 131
D_SPATIAL = 64
D_OUT = 256

NC, NS = 2, 16
NW = NC * NS
B_PER_W = 3200
N_IDX = NW * B_PER_W
CHUNK = 40
SLICES = D_OUT // 16

TC_BLOCK = 512


def _sc_gather_mean(table, nb_flat):
    mesh = plsc.VectorSubcoreMesh(core_axis_name="c", subcore_axis_name="s")

    @functools.partial(
        pl.kernel,
        out_type=jax.ShapeDtypeStruct((N_NODES, D_OUT), jnp.float32),
        mesh=mesh,
        scratch_types=[
            pltpu.VMEM((B_PER_W,), jnp.int32),
            pltpu.VMEM((B_PER_W,), jnp.int32),
            pltpu.VMEM((B_PER_W,), jnp.int32),
            [pltpu.VMEM((CHUNK, D_OUT), jnp.float32) for _ in range(2)],
            [pltpu.VMEM((CHUNK, D_OUT), jnp.float32) for _ in range(2)],
            [pltpu.VMEM((CHUNK, D_OUT), jnp.float32) for _ in range(2)],
            [pltpu.VMEM((CHUNK, D_OUT), jnp.float32) for _ in range(2)],
            [pltpu.SemaphoreType.DMA for _ in range(2)],
            [pltpu.SemaphoreType.DMA for _ in range(2)],
        ],
    )
    def k(table_hbm, nb_hbm, out_hbm,
          i0_v, i1_v, i2_v, g0, g1, g2, acc, sems, osems):
        wid = lax.axis_index("s") * NC + lax.axis_index("c")
        wbase = wid * B_PER_W
        last = wid == NW - 1
        n_chunks = jnp.where(last, 800 // CHUNK, B_PER_W // CHUNK)

        @pl.when(jnp.logical_not(last))
        def _():
            for kk, iv in enumerate((i0_v, i1_v, i2_v)):
                pltpu.sync_copy(
                    nb_hbm.at[pl.ds(kk * N_NODES + wbase, B_PER_W)], iv)

        @pl.when(last)
        def _():
            for kk, iv in enumerate((i0_v, i1_v, i2_v)):
                pltpu.sync_copy(
                    nb_hbm.at[pl.ds(kk * N_NODES + wbase, 800)],
                    iv.at[pl.ds(0, 800)])

        def issue4(c, b):
            lbase = c * CHUNK
            ds = pltpu.async_copy(
                table_hbm.at[pl.ds(wbase + lbase, CHUNK)], acc[b], sems[b])
            dg = []
            for gk, iv in zip((g0, g1, g2), (i0_v, i1_v, i2_v)):
                dg.append(pltpu.async_copy(
                    table_hbm.at[iv.at[pl.ds(lbase, 24)]],
                    gk[b].at[pl.ds(0, 24)], sems[b]))
                dg.append(pltpu.async_copy(
                    table_hbm.at[iv.at[pl.ds(lbase + 24, 16)]],
                    gk[b].at[pl.ds(24, 16)], sems[b]))
            return [ds] + dg

        def compute_and_store(c, b):
            def row_body(j, c2):
                for d in range(SLICES):
                    sl = pl.ds(d * 16, 16)
                    acc[b][j, sl] = (
                        acc[b][j, sl] + g0[b][j, sl] + g1[b][j, sl]
                        + g2[b][j, sl]
                    ) * 0.25
                return c2

            lax.fori_loop(0, CHUNK, row_body, 0, unroll=2)
            pltpu.async_copy(
                acc[b], out_hbm.at[pl.ds(wbase + c * CHUNK, CHUNK)], osems[b])

        def wait_out(c, b):
            pltpu.make_async_copy(
                acc[b], out_hbm.at[pl.ds(wbase + c * CHUNK, CHUNK)],
                osems[b]).wait()

        def pair_body(i, carry):
            c0 = 2 * i

            @pl.when(c0 >= 2)
            def _():
                wait_out(c0 - 2, 0)

            da = issue4(c0, 0)

            @pl.when(c0 >= 2)
            def _():
                wait_out(c0 - 1, 1)

            db = issue4(c0 + 1, 1)
            for d in da:
                d.wait()
            compute_and_store(c0, 0)
            for d in db:
                d.wait()
            compute_and_store(c0 + 1, 1)
            return carry

        lax.fori_loop(0, n_chunks // 2, pair_body, 0)
        wait_out(n_chunks - 2, 0)
        wait_out(n_chunks - 1, 1)

    return k(table, nb_flat)


def _p_body(stt_ref, wa_ref, ba_ref, p_ref):
    p_ref[...] = (
        lax.dot_general(
            stt_ref[...], wa_ref[...],
            dimension_numbers=(((0,), (0,)), ((), ())),
            preferred_element_type=jnp.float32,
        )
        + ba_ref[...]
    )


def _tc_p(structural_t, WaT, b_agg):
    grid = (pl.cdiv(N_NODES, TC_BLOCK),)
    full = lambda i: (0, 0)
    return pl.pallas_call(
        _p_body,
        grid=grid,
        in_specs=[
            pl.BlockSpec((D_STRUCT, TC_BLOCK), lambda i: (0, i)),
            pl.BlockSpec((D_STRUCT, D_OUT), full),
            pl.BlockSpec((1, D_OUT), full),
        ],
        out_specs=pl.BlockSpec((TC_BLOCK, D_OUT), lambda i: (i, 0)),
        out_shape=jax.ShapeDtypeStruct((N_NODES, D_OUT), jnp.float32),
    )(structural_t, WaT, b_agg)


def _out1_body(spt_ref, stt_ref, wc_ref, bc_ref, o1_ref):
    dn = (((0,), (0,)), ((), ()))
    wc = wc_ref[...]
    o1_ref[...] = (
        lax.dot_general(spt_ref[...], wc[:D_SPATIAL], dimension_numbers=dn,
                        preferred_element_type=jnp.float32)
        + lax.dot_general(stt_ref[...], wc[D_SPATIAL:], dimension_numbers=dn,
                          preferred_element_type=jnp.float32)
        + bc_ref[...]
    )


def _tc_out1(spatial_t, structural_t, WcT, b_comb):
    grid = (pl.cdiv(N_NODES, TC_BLOCK),)
    full = lambda i: (0, 0)
    return pl.pallas_call(
        _out1_body,
        grid=grid,
        in_specs=[
            pl.BlockSpec((D_SPATIAL, TC_BLOCK), lambda i: (0, i)),
            pl.BlockSpec((D_STRUCT, TC_BLOCK), lambda i: (0, i)),
            pl.BlockSpec((D_SPATIAL + D_STRUCT, D_OUT), full),
            pl.BlockSpec((1, D_OUT), full),
        ],
        out_specs=pl.BlockSpec((TC_BLOCK, D_OUT), lambda i: (i, 0)),
        out_shape=jax.ShapeDtypeStruct((N_NODES, D_OUT), jnp.float32),
    )(spatial_t, structural_t, WcT, b_comb)


def kernel(spatial, structural, neighbour, W_comb, b_comb, W_agg, b_agg):
    nb_flat = neighbour.astype(jnp.int32).T.reshape(-1)

    WcT = W_comb.T
    WaT = W_agg.T
    spatial_t = spatial.T
    structural_t = structural.T

    P = _tc_p(structural_t, WaT, b_agg.reshape(1, D_OUT))
    out2 = _sc_gather_mean(P, nb_flat)
    out1 = _tc_out1(spatial_t, structural_t, WcT, b_comb.reshape(1, D_OUT))
    return (out1, out2)


# speedup vs baseline: 1.3966x; 1.1148x over previous
import functools

import jax
import jax.numpy as jnp
from jax import lax
from jax.experimental import pallas as pl
from jax.experimental.pallas import tpu as pltpu
from jax.experimental.pallas import tpu_sc as plsc

N_NODES = 100000
D_STRUCT = 131
D_SPATIAL = 64
D_OUT = 256

NC, NS = 2, 16
NW = NC * NS
B_PER_W = 3200
N_IDX = NW * B_PER_W
CHUNK = 40
SLICES = D_OUT // 16

TC_BLOCK = 1024


def _sc_gather_mean(table, nb_flat):
    mesh = plsc.VectorSubcoreMesh(core_axis_name="c", subcore_axis_name="s")

    @functools.partial(
        pl.kernel,
        out_type=jax.ShapeDtypeStruct((N_NODES, D_OUT), jnp.float32),
        mesh=mesh,
        scratch_types=[
            pltpu.VMEM((B_PER_W,), jnp.int32),
            pltpu.VMEM((B_PER_W,), jnp.int32),
            pltpu.VMEM((B_PER_W,), jnp.int32),
            [pltpu.VMEM((CHUNK, D_OUT), jnp.float32) for _ in range(2)],
            [pltpu.VMEM((CHUNK, D_OUT), jnp.float32) for _ in range(2)],
            [pltpu.VMEM((CHUNK, D_OUT), jnp.float32) for _ in range(2)],
            [pltpu.VMEM((CHUNK, D_OUT), jnp.float32) for _ in range(2)],
            [pltpu.SemaphoreType.DMA for _ in range(2)],
            [pltpu.SemaphoreType.DMA for _ in range(2)],
        ],
    )
    def k(table_hbm, nb_hbm, out_hbm,
          i0_v, i1_v, i2_v, g0, g1, g2, acc, sems, osems):
        wid = lax.axis_index("s") * NC + lax.axis_index("c")
        wbase = wid * B_PER_W
        last = wid == NW - 1
        n_chunks = jnp.where(last, 800 // CHUNK, B_PER_W // CHUNK)

        @pl.when(jnp.logical_not(last))
        def _():
            for kk, iv in enumerate((i0_v, i1_v, i2_v)):
                pltpu.sync_copy(
                    nb_hbm.at[pl.ds(kk * N_NODES + wbase, B_PER_W)], iv)

        @pl.when(last)
        def _():
            for kk, iv in enumerate((i0_v, i1_v, i2_v)):
                pltpu.sync_copy(
                    nb_hbm.at[pl.ds(kk * N_NODES + wbase, 800)],
                    iv.at[pl.ds(0, 800)])

        def issue4(c, b):
            lbase = c * CHUNK
            ds = pltpu.async_copy(
                table_hbm.at[pl.ds(wbase + lbase, CHUNK)], acc[b], sems[b])
            dg = []
            for gk, iv in zip((g0, g1, g2), (i0_v, i1_v, i2_v)):
                dg.append(pltpu.async_copy(
                    table_hbm.at[iv.at[pl.ds(lbase, 24)]],
                    gk[b].at[pl.ds(0, 24)], sems[b]))
                dg.append(pltpu.async_copy(
                    table_hbm.at[iv.at[pl.ds(lbase + 24, 16)]],
                    gk[b].at[pl.ds(24, 16)], sems[b]))
            return [ds] + dg

        def compute_and_store(c, b):
            def row_body(j, c2):
                for d in range(SLICES):
                    sl = pl.ds(d * 16, 16)
                    acc[b][j, sl] = (
                        acc[b][j, sl] + g0[b][j, sl] + g1[b][j, sl]
                        + g2[b][j, sl]
                    ) * 0.25
                return c2

            lax.fori_loop(0, CHUNK, row_body, 0, unroll=2)
            pltpu.async_copy(
                acc[b], out_hbm.at[pl.ds(wbase + c * CHUNK, CHUNK)], osems[b])

        def wait_out(c, b):
            pltpu.make_async_copy(
                acc[b], out_hbm.at[pl.ds(wbase + c * CHUNK, CHUNK)],
                osems[b]).wait()

        def pair_body(i, carry):
            c0 = 2 * i

            @pl.when(c0 >= 2)
            def _():
                wait_out(c0 - 2, 0)

            da = issue4(c0, 0)

            @pl.when(c0 >= 2)
            def _():
                wait_out(c0 - 1, 1)

            db = issue4(c0 + 1, 1)
            for d in da:
                d.wait()
            compute_and_store(c0, 0)
            for d in db:
                d.wait()
            compute_and_store(c0 + 1, 1)
            return carry

        lax.fori_loop(0, n_chunks // 2, pair_body, 0)
        wait_out(n_chunks - 2, 0)
        wait_out(n_chunks - 1, 1)

    return k(table, nb_flat)


def _p_body(stt_ref, wa_ref, ba_ref, p_ref):
    p_ref[...] = (
        lax.dot_general(
            stt_ref[...], wa_ref[...],
            dimension_numbers=(((0,), (0,)), ((), ())),
            preferred_element_type=jnp.float32,
        )
        + ba_ref[...]
    )


def _tc_p(structural_t, WaT, b_agg):
    grid = (pl.cdiv(N_NODES, TC_BLOCK),)
    full = lambda i: (0, 0)
    return pl.pallas_call(
        _p_body,
        grid=grid,
        in_specs=[
            pl.BlockSpec((D_STRUCT, TC_BLOCK), lambda i: (0, i)),
            pl.BlockSpec((D_STRUCT, D_OUT), full),
            pl.BlockSpec((1, D_OUT), full),
        ],
        out_specs=pl.BlockSpec((TC_BLOCK, D_OUT), lambda i: (i, 0)),
        out_shape=jax.ShapeDtypeStruct((N_NODES, D_OUT), jnp.float32),
    )(structural_t, WaT, b_agg)


def _out1_body(spt_ref, stt_ref, wc_ref, bc_ref, o1_ref):
    dn = (((0,), (0,)), ((), ()))
    wc = wc_ref[...]
    o1_ref[...] = (
        lax.dot_general(spt_ref[...], wc[:D_SPATIAL], dimension_numbers=dn,
                        preferred_element_type=jnp.float32)
        + lax.dot_general(stt_ref[...], wc[D_SPATIAL:], dimension_numbers=dn,
                          preferred_element_type=jnp.float32)
        + bc_ref[...]
    )


def _tc_out1(spatial_t, structural_t, WcT, b_comb):
    grid = (pl.cdiv(N_NODES, TC_BLOCK),)
    full = lambda i: (0, 0)
    return pl.pallas_call(
        _out1_body,
        grid=grid,
        in_specs=[
            pl.BlockSpec((D_SPATIAL, TC_BLOCK), lambda i: (0, i)),
            pl.BlockSpec((D_STRUCT, TC_BLOCK), lambda i: (0, i)),
            pl.BlockSpec((D_SPATIAL + D_STRUCT, D_OUT), full),
            pl.BlockSpec((1, D_OUT), full),
        ],
        out_specs=pl.BlockSpec((TC_BLOCK, D_OUT), lambda i: (i, 0)),
        out_shape=jax.ShapeDtypeStruct((N_NODES, D_OUT), jnp.float32),
    )(spatial_t, structural_t, WcT, b_comb)


def kernel(spatial, structural, neighbour, W_comb, b_comb, W_agg, b_agg):
    nb_flat = neighbour.astype(jnp.int32).T.reshape(-1)

    WcT = W_comb.T
    WaT = W_agg.T
    spatial_t = spatial.T
    structural_t = structural.T

    P = _tc_p(structural_t, WaT, b_agg.reshape(1, D_OUT))
    out2 = _sc_gather_mean(P, nb_flat)
    out1 = _tc_out1(spatial_t, structural_t, WcT, b_comb.reshape(1, D_OUT))
    return (out1, out2)


# speedup vs baseline: 1.4619x; 1.0468x over previous
import functools

import jax
import jax.numpy as jnp
from jax import lax
from jax.experimental import pallas as pl
from jax.experimental.pallas import tpu as pltpu
from jax.experimental.pallas import tpu_sc as plsc

N_NODES = 100000
D_STRUCT = 131
D_SPATIAL = 64
D_OUT = 256

NC, NS = 2, 16
NW = NC * NS
B_PER_W = 3200
N_IDX = NW * B_PER_W
CHUNK = 40
SLICES = D_OUT // 16

TC_BLOCK = 2048


def _sc_gather_mean(table, nb_flat):
    mesh = plsc.VectorSubcoreMesh(core_axis_name="c", subcore_axis_name="s")

    @functools.partial(
        pl.kernel,
        out_type=jax.ShapeDtypeStruct((N_NODES, D_OUT), jnp.float32),
        mesh=mesh,
        scratch_types=[
            pltpu.VMEM((B_PER_W,), jnp.int32),
            pltpu.VMEM((B_PER_W,), jnp.int32),
            pltpu.VMEM((B_PER_W,), jnp.int32),
            [pltpu.VMEM((CHUNK, D_OUT), jnp.float32) for _ in range(2)],
            [pltpu.VMEM((CHUNK, D_OUT), jnp.float32) for _ in range(2)],
            [pltpu.VMEM((CHUNK, D_OUT), jnp.float32) for _ in range(2)],
            [pltpu.VMEM((CHUNK, D_OUT), jnp.float32) for _ in range(2)],
            [pltpu.SemaphoreType.DMA for _ in range(2)],
            [pltpu.SemaphoreType.DMA for _ in range(2)],
        ],
    )
    def k(table_hbm, nb_hbm, out_hbm,
          i0_v, i1_v, i2_v, g0, g1, g2, acc, sems, osems):
        wid = lax.axis_index("s") * NC + lax.axis_index("c")
        wbase = wid * B_PER_W
        last = wid == NW - 1
        n_chunks = jnp.where(last, 800 // CHUNK, B_PER_W // CHUNK)

        @pl.when(jnp.logical_not(last))
        def _():
            for kk, iv in enumerate((i0_v, i1_v, i2_v)):
                pltpu.sync_copy(
                    nb_hbm.at[pl.ds(kk * N_NODES + wbase, B_PER_W)], iv)

        @pl.when(last)
        def _():
            for kk, iv in enumerate((i0_v, i1_v, i2_v)):
                pltpu.sync_copy(
                    nb_hbm.at[pl.ds(kk * N_NODES + wbase, 800)],
                    iv.at[pl.ds(0, 800)])

        def issue4(c, b):
            lbase = c * CHUNK
            ds = pltpu.async_copy(
                table_hbm.at[pl.ds(wbase + lbase, CHUNK)], acc[b], sems[b])
            dg = []
            for gk, iv in zip((g0, g1, g2), (i0_v, i1_v, i2_v)):
                dg.append(pltpu.async_copy(
                    table_hbm.at[iv.at[pl.ds(lbase, 24)]],
                    gk[b].at[pl.ds(0, 24)], sems[b]))
                dg.append(pltpu.async_copy(
                    table_hbm.at[iv.at[pl.ds(lbase + 24, 16)]],
                    gk[b].at[pl.ds(24, 16)], sems[b]))
            return [ds] + dg

        def compute_and_store(c, b):
            def row_body(j, c2):
                for d in range(SLICES):
                    sl = pl.ds(d * 16, 16)
                    acc[b][j, sl] = (
                        acc[b][j, sl] + g0[b][j, sl] + g1[b][j, sl]
                        + g2[b][j, sl]
                    ) * 0.25
                return c2

            lax.fori_loop(0, CHUNK, row_body, 0, unroll=2)
            pltpu.async_copy(
                acc[b], out_hbm.at[pl.ds(wbase + c * CHUNK, CHUNK)], osems[b])

        def wait_out(c, b):
            pltpu.make_async_copy(
                acc[b], out_hbm.at[pl.ds(wbase + c * CHUNK, CHUNK)],
                osems[b]).wait()

        def pair_body(i, carry):
            c0 = 2 * i

            @pl.when(c0 >= 2)
            def _():
                wait_out(c0 - 2, 0)

            da = issue4(c0, 0)

            @pl.when(c0 >= 2)
            def _():
                wait_out(c0 - 1, 1)

            db = issue4(c0 + 1, 1)
            for d in da:
                d.wait()
            compute_and_store(c0, 0)
            for d in db:
                d.wait()
            compute_and_store(c0 + 1, 1)
            return carry

        lax.fori_loop(0, n_chunks // 2, pair_body, 0)
        wait_out(n_chunks - 2, 0)
        wait_out(n_chunks - 1, 1)

    return k(table, nb_flat)


def _p_body(stt_ref, wa_ref, ba_ref, p_ref):
    p_ref[...] = (
        lax.dot_general(
            stt_ref[...], wa_ref[...],
            dimension_numbers=(((0,), (0,)), ((), ())),
            preferred_element_type=jnp.float32,
        )
        + ba_ref[...]
    )


def _tc_p(structural_t, WaT, b_agg):
    grid = (pl.cdiv(N_NODES, TC_BLOCK),)
    full = lambda i: (0, 0)
    return pl.pallas_call(
        _p_body,
        grid=grid,
        in_specs=[
            pl.BlockSpec((D_STRUCT, TC_BLOCK), lambda i: (0, i)),
            pl.BlockSpec((D_STRUCT, D_OUT), full),
            pl.BlockSpec((1, D_OUT), full),
        ],
        out_specs=pl.BlockSpec((TC_BLOCK, D_OUT), lambda i: (i, 0)),
        out_shape=jax.ShapeDtypeStruct((N_NODES, D_OUT), jnp.float32),
    )(structural_t, WaT, b_agg)


def _out1_body(spt_ref, stt_ref, wc_ref, bc_ref, o1_ref):
    dn = (((0,), (0,)), ((), ()))
    wc = wc_ref[...]
    o1_ref[...] = (
        lax.dot_general(spt_ref[...], wc[:D_SPATIAL], dimension_numbers=dn,
                        preferred_element_type=jnp.float32)
        + lax.dot_general(stt_ref[...], wc[D_SPATIAL:], dimension_numbers=dn,
                          preferred_element_type=jnp.float32)
        + bc_ref[...]
    )


def _tc_out1(spatial_t, structural_t, WcT, b_comb):
    grid = (pl.cdiv(N_NODES, TC_BLOCK),)
    full = lambda i: (0, 0)
    return pl.pallas_call(
        _out1_body,
        grid=grid,
        in_specs=[
            pl.BlockSpec((D_SPATIAL, TC_BLOCK), lambda i: (0, i)),
            pl.BlockSpec((D_STRUCT, TC_BLOCK), lambda i: (0, i)),
            pl.BlockSpec((D_SPATIAL + D_STRUCT, D_OUT), full),
            pl.BlockSpec((1, D_OUT), full),
        ],
        out_specs=pl.BlockSpec((TC_BLOCK, D_OUT), lambda i: (i, 0)),
        out_shape=jax.ShapeDtypeStruct((N_NODES, D_OUT), jnp.float32),
    )(spatial_t, structural_t, WcT, b_comb)


def kernel(spatial, structural, neighbour, W_comb, b_comb, W_agg, b_agg):
    nb_flat = neighbour.astype(jnp.int32).T.reshape(-1)

    WcT = W_comb.T
    WaT = W_agg.T
    spatial_t = spatial.T
    structural_t = structural.T

    P = _tc_p(structural_t, WaT, b_agg.reshape(1, D_OUT))
    out2 = _sc_gather_mean(P, nb_flat)
    out1 = _tc_out1(spatial_t, structural_t, WcT, b_comb.reshape(1, D_OUT))
    return (out1, out2)


# speedup vs baseline: 1.4794x; 1.0120x over previous
import functools

import jax
import jax.numpy as jnp
from jax import lax
from jax.experimental import pallas as pl
from jax.experimental.pallas import tpu as pltpu
from jax.experimental.pallas import tpu_sc as plsc

N_NODES = 100000
D_STRUCT = 131
D_SPATIAL = 64
D_OUT = 256

NC, NS = 2, 16
NW = NC * NS
B_PER_W = 3200
N_IDX = NW * B_PER_W
CHUNK = 40
SLICES = D_OUT // 16

TC_BLOCK = 4096


def _sc_gather_mean(table, nb_flat):
    mesh = plsc.VectorSubcoreMesh(core_axis_name="c", subcore_axis_name="s")

    @functools.partial(
        pl.kernel,
        out_type=jax.ShapeDtypeStruct((N_NODES, D_OUT), jnp.float32),
        mesh=mesh,
        scratch_types=[
            pltpu.VMEM((B_PER_W,), jnp.int32),
            pltpu.VMEM((B_PER_W,), jnp.int32),
            pltpu.VMEM((B_PER_W,), jnp.int32),
            [pltpu.VMEM((CHUNK, D_OUT), jnp.float32) for _ in range(2)],
            [pltpu.VMEM((CHUNK, D_OUT), jnp.float32) for _ in range(2)],
            [pltpu.VMEM((CHUNK, D_OUT), jnp.float32) for _ in range(2)],
            [pltpu.VMEM((CHUNK, D_OUT), jnp.float32) for _ in range(2)],
            [pltpu.SemaphoreType.DMA for _ in range(2)],
            [pltpu.SemaphoreType.DMA for _ in range(2)],
        ],
    )
    def k(table_hbm, nb_hbm, out_hbm,
          i0_v, i1_v, i2_v, g0, g1, g2, acc, sems, osems):
        wid = lax.axis_index("s") * NC + lax.axis_index("c")
        wbase = wid * B_PER_W
        last = wid == NW - 1
        n_chunks = jnp.where(last, 800 // CHUNK, B_PER_W // CHUNK)

        @pl.when(jnp.logical_not(last))
        def _():
            for kk, iv in enumerate((i0_v, i1_v, i2_v)):
                pltpu.sync_copy(
                    nb_hbm.at[pl.ds(kk * N_NODES + wbase, B_PER_W)], iv)

        @pl.when(last)
        def _():
            for kk, iv in enumerate((i0_v, i1_v, i2_v)):
                pltpu.sync_copy(
                    nb_hbm.at[pl.ds(kk * N_NODES + wbase, 800)],
                    iv.at[pl.ds(0, 800)])

        def issue4(c, b):
            lbase = c * CHUNK
            ds = pltpu.async_copy(
                table_hbm.at[pl.ds(wbase + lbase, CHUNK)], acc[b], sems[b])
            dg = []
            for gk, iv in zip((g0, g1, g2), (i0_v, i1_v, i2_v)):
                dg.append(pltpu.async_copy(
                    table_hbm.at[iv.at[pl.ds(lbase, 24)]],
                    gk[b].at[pl.ds(0, 24)], sems[b]))
                dg.append(pltpu.async_copy(
                    table_hbm.at[iv.at[pl.ds(lbase + 24, 16)]],
                    gk[b].at[pl.ds(24, 16)], sems[b]))
            return [ds] + dg

        def compute_and_store(c, b):
            def row_body(j, c2):
                for d in range(SLICES):
                    sl = pl.ds(d * 16, 16)
                    acc[b][j, sl] = (
                        acc[b][j, sl] + g0[b][j, sl] + g1[b][j, sl]
                        + g2[b][j, sl]
                    ) * 0.25
                return c2

            lax.fori_loop(0, CHUNK, row_body, 0, unroll=2)
            pltpu.async_copy(
                acc[b], out_hbm.at[pl.ds(wbase + c * CHUNK, CHUNK)], osems[b])

        def wait_out(c, b):
            pltpu.make_async_copy(
                acc[b], out_hbm.at[pl.ds(wbase + c * CHUNK, CHUNK)],
                osems[b]).wait()

        def pair_body(i, carry):
            c0 = 2 * i

            @pl.when(c0 >= 2)
            def _():
                wait_out(c0 - 2, 0)

            da = issue4(c0, 0)

            @pl.when(c0 >= 2)
            def _():
                wait_out(c0 - 1, 1)

            db = issue4(c0 + 1, 1)
            for d in da:
                d.wait()
            compute_and_store(c0, 0)
            for d in db:
                d.wait()
            compute_and_store(c0 + 1, 1)
            return carry

        lax.fori_loop(0, n_chunks // 2, pair_body, 0)
        wait_out(n_chunks - 2, 0)
        wait_out(n_chunks - 1, 1)

    return k(table, nb_flat)


def _p_body(stt_ref, wa_ref, ba_ref, p_ref):
    p_ref[...] = (
        lax.dot_general(
            stt_ref[...], wa_ref[...],
            dimension_numbers=(((0,), (0,)), ((), ())),
            preferred_element_type=jnp.float32,
        )
        + ba_ref[...]
    )


def _tc_p(structural_t, WaT, b_agg):
    grid = (pl.cdiv(N_NODES, TC_BLOCK),)
    full = lambda i: (0, 0)
    return pl.pallas_call(
        _p_body,
        grid=grid,
        in_specs=[
            pl.BlockSpec((D_STRUCT, TC_BLOCK), lambda i: (0, i)),
            pl.BlockSpec((D_STRUCT, D_OUT), full),
            pl.BlockSpec((1, D_OUT), full),
        ],
        out_specs=pl.BlockSpec((TC_BLOCK, D_OUT), lambda i: (i, 0)),
        out_shape=jax.ShapeDtypeStruct((N_NODES, D_OUT), jnp.float32),
    )(structural_t, WaT, b_agg)


def _out1_body(spt_ref, stt_ref, wc_ref, bc_ref, o1_ref):
    dn = (((0,), (0,)), ((), ()))
    wc = wc_ref[...]
    o1_ref[...] = (
        lax.dot_general(spt_ref[...], wc[:D_SPATIAL], dimension_numbers=dn,
                        preferred_element_type=jnp.float32)
        + lax.dot_general(stt_ref[...], wc[D_SPATIAL:], dimension_numbers=dn,
                          preferred_element_type=jnp.float32)
        + bc_ref[...]
    )


def _tc_out1(spatial_t, structural_t, WcT, b_comb):
    grid = (pl.cdiv(N_NODES, TC_BLOCK),)
    full = lambda i: (0, 0)
    return pl.pallas_call(
        _out1_body,
        grid=grid,
        in_specs=[
            pl.BlockSpec((D_SPATIAL, TC_BLOCK), lambda i: (0, i)),
            pl.BlockSpec((D_STRUCT, TC_BLOCK), lambda i: (0, i)),
            pl.BlockSpec((D_SPATIAL + D_STRUCT, D_OUT), full),
            pl.BlockSpec((1, D_OUT), full),
        ],
        out_specs=pl.BlockSpec((TC_BLOCK, D_OUT), lambda i: (i, 0)),
        out_shape=jax.ShapeDtypeStruct((N_NODES, D_OUT), jnp.float32),
    )(spatial_t, structural_t, WcT, b_comb)


def kernel(spatial, structural, neighbour, W_comb, b_comb, W_agg, b_agg):
    nb_flat = neighbour.astype(jnp.int32).T.reshape(-1)

    WcT = W_comb.T
    WaT = W_agg.T
    spatial_t = spatial.T
    structural_t = structural.T

    P = _tc_p(structural_t, WaT, b_agg.reshape(1, D_OUT))
    out2 = _sc_gather_mean(P, nb_flat)
    out1 = _tc_out1(spatial_t, structural_t, WcT, b_comb.reshape(1, D_OUT))
    return (out1, out2)


# speedup vs baseline: 1.4859x; 1.0044x over previous
import functools

import jax
import jax.numpy as jnp
from jax import lax
from jax.experimental import pallas as pl
from jax.experimental.pallas import tpu as pltpu
from jax.experimental.pallas import tpu_sc as plsc

N_NODES = 100000
D_STRUCT = 131
D_SPATIAL = 64
D_OUT = 256

NC, NS = 2, 16
NW = NC * NS
B_PER_W = 3200
N_IDX = NW * B_PER_W
CHUNK = 40
SLICES = D_OUT // 16

TC_BLOCK = 8192


def _sc_gather_mean(table, nb_flat):
    mesh = plsc.VectorSubcoreMesh(core_axis_name="c", subcore_axis_name="s")

    @functools.partial(
        pl.kernel,
        out_type=jax.ShapeDtypeStruct((N_NODES, D_OUT), jnp.float32),
        mesh=mesh,
        scratch_types=[
            pltpu.VMEM((B_PER_W,), jnp.int32),
            pltpu.VMEM((B_PER_W,), jnp.int32),
            pltpu.VMEM((B_PER_W,), jnp.int32),
            [pltpu.VMEM((CHUNK, D_OUT), jnp.float32) for _ in range(2)],
            [pltpu.VMEM((CHUNK, D_OUT), jnp.float32) for _ in range(2)],
            [pltpu.VMEM((CHUNK, D_OUT), jnp.float32) for _ in range(2)],
            [pltpu.VMEM((CHUNK, D_OUT), jnp.float32) for _ in range(2)],
            [pltpu.SemaphoreType.DMA for _ in range(2)],
            [pltpu.SemaphoreType.DMA for _ in range(2)],
        ],
    )
    def k(table_hbm, nb_hbm, out_hbm,
          i0_v, i1_v, i2_v, g0, g1, g2, acc, sems, osems):
        wid = lax.axis_index("s") * NC + lax.axis_index("c")
        wbase = wid * B_PER_W
        last = wid == NW - 1
        n_chunks = jnp.where(last, 800 // CHUNK, B_PER_W // CHUNK)

        @pl.when(jnp.logical_not(last))
        def _():
            for kk, iv in enumerate((i0_v, i1_v, i2_v)):
                pltpu.sync_copy(
                    nb_hbm.at[pl.ds(kk * N_NODES + wbase, B_PER_W)], iv)

        @pl.when(last)
        def _():
            for kk, iv in enumerate((i0_v, i1_v, i2_v)):
                pltpu.sync_copy(
                    nb_hbm.at[pl.ds(kk * N_NODES + wbase, 800)],
                    iv.at[pl.ds(0, 800)])

        def issue4(c, b):
            lbase = c * CHUNK
            ds = pltpu.async_copy(
                table_hbm.at[pl.ds(wbase + lbase, CHUNK)], acc[b], sems[b])
            dg = []
            for gk, iv in zip((g0, g1, g2), (i0_v, i1_v, i2_v)):
                dg.append(pltpu.async_copy(
                    table_hbm.at[iv.at[pl.ds(lbase, 24)]],
                    gk[b].at[pl.ds(0, 24)], sems[b]))
                dg.append(pltpu.async_copy(
                    table_hbm.at[iv.at[pl.ds(lbase + 24, 16)]],
                    gk[b].at[pl.ds(24, 16)], sems[b]))
            return [ds] + dg

        def compute_and_store(c, b):
            def row_body(j, c2):
                for d in range(SLICES):
                    sl = pl.ds(d * 16, 16)
                    acc[b][j, sl] = (
                        acc[b][j, sl] + g0[b][j, sl] + g1[b][j, sl]
                        + g2[b][j, sl]
                    ) * 0.25
                return c2

            lax.fori_loop(0, CHUNK, row_body, 0, unroll=2)
            pltpu.async_copy(
                acc[b], out_hbm.at[pl.ds(wbase + c * CHUNK, CHUNK)], osems[b])

        def wait_out(c, b):
            pltpu.make_async_copy(
                acc[b], out_hbm.at[pl.ds(wbase + c * CHUNK, CHUNK)],
                osems[b]).wait()

        def pair_body(i, carry):
            c0 = 2 * i

            @pl.when(c0 >= 2)
            def _():
                wait_out(c0 - 2, 0)

            da = issue4(c0, 0)

            @pl.when(c0 >= 2)
            def _():
                wait_out(c0 - 1, 1)

            db = issue4(c0 + 1, 1)
            for d in da:
                d.wait()
            compute_and_store(c0, 0)
            for d in db:
                d.wait()
            compute_and_store(c0 + 1, 1)
            return carry

        lax.fori_loop(0, n_chunks // 2, pair_body, 0)
        wait_out(n_chunks - 2, 0)
        wait_out(n_chunks - 1, 1)

    return k(table, nb_flat)


def _p_body(stt_ref, wa_ref, ba_ref, p_ref):
    p_ref[...] = (
        lax.dot_general(
            stt_ref[...], wa_ref[...],
            dimension_numbers=(((0,), (0,)), ((), ())),
            preferred_element_type=jnp.float32,
        )
        + ba_ref[...]
    )


def _tc_p(structural_t, WaT, b_agg):
    grid = (pl.cdiv(N_NODES, TC_BLOCK),)
    full = lambda i: (0, 0)
    return pl.pallas_call(
        _p_body,
        grid=grid,
        in_specs=[
            pl.BlockSpec((D_STRUCT, TC_BLOCK), lambda i: (0, i)),
            pl.BlockSpec((D_STRUCT, D_OUT), full),
            pl.BlockSpec((1, D_OUT), full),
        ],
        out_specs=pl.BlockSpec((TC_BLOCK, D_OUT), lambda i: (i, 0)),
        out_shape=jax.ShapeDtypeStruct((N_NODES, D_OUT), jnp.float32),
    )(structural_t, WaT, b_agg)


def _out1_body(spt_ref, stt_ref, wc_ref, bc_ref, o1_ref):
    dn = (((0,), (0,)), ((), ()))
    wc = wc_ref[...]
    o1_ref[...] = (
        lax.dot_general(spt_ref[...], wc[:D_SPATIAL], dimension_numbers=dn,
                        preferred_element_type=jnp.float32)
        + lax.dot_general(stt_ref[...], wc[D_SPATIAL:], dimension_numbers=dn,
                          preferred_element_type=jnp.float32)
        + bc_ref[...]
    )


def _tc_out1(spatial_t, structural_t, WcT, b_comb):
    grid = (pl.cdiv(N_NODES, TC_BLOCK),)
    full = lambda i: (0, 0)
    return pl.pallas_call(
        _out1_body,
        grid=grid,
        in_specs=[
            pl.BlockSpec((D_SPATIAL, TC_BLOCK), lambda i: (0, i)),
            pl.BlockSpec((D_STRUCT, TC_BLOCK), lambda i: (0, i)),
            pl.BlockSpec((D_SPATIAL + D_STRUCT, D_OUT), full),
            pl.BlockSpec((1, D_OUT), full),
        ],
        out_specs=pl.BlockSpec((TC_BLOCK, D_OUT), lambda i: (i, 0)),
        out_shape=jax.ShapeDtypeStruct((N_NODES, D_OUT), jnp.float32),
    )(spatial_t, structural_t, WcT, b_comb)


def kernel(spatial, structural, neighbour, W_comb, b_comb, W_agg, b_agg):
    nb_flat = neighbour.astype(jnp.int32).T.reshape(-1)

    WcT = W_comb.T
    WaT = W_agg.T
    spatial_t = spatial.T
    structural_t = structural.T

    P = _tc_p(structural_t, WaT, b_agg.reshape(1, D_OUT))
    out2 = _sc_gather_mean(P, nb_flat)
    out1 = _tc_out1(spatial_t, structural_t, WcT, b_comb.reshape(1, D_OUT))
    return (out1, out2)


# speedup vs baseline: 1.4896x; 1.0025x over previous
import functools

import jax
import jax.numpy as jnp
from jax import lax
from jax.experimental import pallas as pl
from jax.experimental.pallas import tpu as pltpu
from jax.experimental.pallas import tpu_sc as plsc

N_NODES = 100000
D_STRUCT = 131
D_SPATIAL = 64
D_OUT = 256

NC, NS = 2, 16
NW = NC * NS
B_PER_W = 3200
CHUNK = 40
SLICES = D_OUT // 16

TC_BLOCK = 8192


def _sc_gather_mean(table, nb_flat):
    mesh = plsc.VectorSubcoreMesh(core_axis_name="c", subcore_axis_name="s")

    @functools.partial(
        pl.kernel,
        out_type=jax.ShapeDtypeStruct((N_NODES, D_OUT), jnp.float32),
        mesh=mesh,
        scratch_types=[
            pltpu.VMEM((B_PER_W,), jnp.int32),
            pltpu.VMEM((B_PER_W,), jnp.int32),
            pltpu.VMEM((B_PER_W,), jnp.int32),
            [pltpu.VMEM((CHUNK, D_OUT), jnp.float32) for _ in range(2)],
            [pltpu.VMEM((CHUNK, D_OUT), jnp.float32) for _ in range(2)],
            [pltpu.VMEM((CHUNK, D_OUT), jnp.float32) for _ in range(2)],
            [pltpu.VMEM((CHUNK, D_OUT), jnp.float32) for _ in range(2)],
            [pltpu.SemaphoreType.DMA for _ in range(2)],
            [pltpu.SemaphoreType.DMA for _ in range(2)],
        ],
    )
    def k(table_hbm, nb_hbm, out_hbm,
          i0_v, i1_v, i2_v, g0, g1, g2, acc, sems, osems):
        wid = lax.axis_index("s") * NC + lax.axis_index("c")
        wbase = wid * B_PER_W
        last = wid == NW - 1
        n_chunks = jnp.where(last, 800 // CHUNK, B_PER_W // CHUNK)

        @pl.when(jnp.logical_not(last))
        def _():
            for kk, iv in enumerate((i0_v, i1_v, i2_v)):
                pltpu.sync_copy(
                    nb_hbm.at[pl.ds(kk * N_NODES + wbase, B_PER_W)], iv)

        @pl.when(last)
        def _():
            for kk, iv in enumerate((i0_v, i1_v, i2_v)):
                pltpu.sync_copy(
                    nb_hbm.at[pl.ds(kk * N_NODES + wbase, 800)],
                    iv.at[pl.ds(0, 800)])

        def issue4(c, b):
            lbase = c * CHUNK
            ds = pltpu.async_copy(
                table_hbm.at[pl.ds(wbase + lbase, CHUNK)], acc[b], sems[b])
            dg = []
            for gk, iv in zip((g0, g1, g2), (i0_v, i1_v, i2_v)):
                dg.append(pltpu.async_copy(
                    table_hbm.at[iv.at[pl.ds(lbase, 24)]],
                    gk[b].at[pl.ds(0, 24)], sems[b]))
                dg.append(pltpu.async_copy(
                    table_hbm.at[iv.at[pl.ds(lbase + 24, 16)]],
                    gk[b].at[pl.ds(24, 16)], sems[b]))
            return [ds] + dg

        def compute_and_store(c, b):
            def row_body(j, c2):
                for d in range(SLICES):
                    sl = pl.ds(d * 16, 16)
                    acc[b][j, sl] = (
                        acc[b][j, sl] + g0[b][j, sl] + g1[b][j, sl]
                        + g2[b][j, sl]
                    ) * 0.25
                return c2

            lax.fori_loop(0, CHUNK, row_body, 0, unroll=2)
            pltpu.async_copy(
                acc[b], out_hbm.at[pl.ds(wbase + c * CHUNK, CHUNK)], osems[b])

        def wait_out(c, b):
            pltpu.make_async_copy(
                acc[b], out_hbm.at[pl.ds(wbase + c * CHUNK, CHUNK)],
                osems[b]).wait()

        def pair_body(i, carry):
            c0 = 2 * i

            @pl.when(c0 >= 2)
            def _():
                wait_out(c0 - 2, 0)

            da = issue4(c0, 0)

            @pl.when(c0 >= 2)
            def _():
                wait_out(c0 - 1, 1)

            db = issue4(c0 + 1, 1)
            for d in da:
                d.wait()
            compute_and_store(c0, 0)
            for d in db:
                d.wait()
            compute_and_store(c0 + 1, 1)
            return carry

        lax.fori_loop(0, n_chunks // 2, pair_body, 0)
        wait_out(n_chunks - 2, 0)
        wait_out(n_chunks - 1, 1)

    return k(table, nb_flat)


def _p_body(stt_ref, wa_ref, ba_ref, p_ref):
    p_ref[...] = (
        lax.dot_general(
            stt_ref[...], wa_ref[...],
            dimension_numbers=(((0,), (0,)), ((), ())),
            preferred_element_type=jnp.float32,
        )
        + ba_ref[...]
    )


def _tc_p(structural_t, WaT, b_agg):
    grid = (pl.cdiv(N_NODES, TC_BLOCK),)
    full = lambda i: (0, 0)
    return pl.pallas_call(
        _p_body,
        grid=grid,
        in_specs=[
            pl.BlockSpec((D_STRUCT, TC_BLOCK), lambda i: (0, i)),
            pl.BlockSpec((D_STRUCT, D_OUT), full),
            pl.BlockSpec((1, D_OUT), full),
        ],
        out_specs=pl.BlockSpec((TC_BLOCK, D_OUT), lambda i: (i, 0)),
        out_shape=jax.ShapeDtypeStruct((N_NODES, D_OUT), jnp.float32),
    )(structural_t, WaT, b_agg)


def _out1_body(spt_ref, stt_ref, wc_ref, bc_ref, o1_ref):
    dn = (((0,), (0,)), ((), ()))
    wc = wc_ref[...]
    o1_ref[...] = (
        lax.dot_general(spt_ref[...], wc[:D_SPATIAL], dimension_numbers=dn,
                        preferred_element_type=jnp.float32)
        + lax.dot_general(stt_ref[...], wc[D_SPATIAL:], dimension_numbers=dn,
                          preferred_element_type=jnp.float32)
        + bc_ref[...]
    )


def _tc_out1(spatial_t, structural_t, WcT, b_comb):
    grid = (pl.cdiv(N_NODES, TC_BLOCK),)
    full = lambda i: (0, 0)
    return pl.pallas_call(
        _out1_body,
        grid=grid,
        in_specs=[
            pl.BlockSpec((D_SPATIAL, TC_BLOCK), lambda i: (0, i)),
            pl.BlockSpec((D_STRUCT, TC_BLOCK), lambda i: (0, i)),
            pl.BlockSpec((D_SPATIAL + D_STRUCT, D_OUT), full),
            pl.BlockSpec((1, D_OUT), full),
        ],
        out_specs=pl.BlockSpec((TC_BLOCK, D_OUT), lambda i: (i, 0)),
        out_shape=jax.ShapeDtypeStruct((N_NODES, D_OUT), jnp.float32),
    )(spatial_t, structural_t, WcT, b_comb)


def kernel(spatial, structural, neighbour, W_comb, b_comb, W_agg, b_agg):
    nb_flat = neighbour.astype(jnp.int32).T.reshape(-1)

    WcT = W_comb.T
    WaT = W_agg.T
    spatial_t = spatial.T
    structural_t = structural.T

    P = _tc_p(structural_t, WaT, b_agg.reshape(1, D_OUT))
    out2 = _sc_gather_mean(P, nb_flat)
    out1 = _tc_out1(spatial_t, structural_t, WcT, b_comb.reshape(1, D_OUT))
    return (out1, out2)
